# baseline (speedup 1.0000x reference)
import functools

import jax
import jax.numpy as jnp
from jax import lax
from jax.experimental import pallas as pl
from jax.experimental.pallas import tpu as pltpu
from jax.experimental.pallas import tpu_sc as plsc

N_NODES = 10000
D_FEAT = 128
NUM_CLASSES = 2
N_EDGES = 320000

NP = 10240
ROWS = NP // 128
NT = 16
NC = 2
EPT = N_EDGES // NT
BLK = 2048


def _project_body(x_ref, w_ref, y_ref):
    y_ref[...] = lax.dot_general(
        w_ref[...], x_ref[...],
        dimension_numbers=(((1,), (1,)), ((), ())),
        preferred_element_type=jnp.float32,
    )


def _project(x, W):
    return pl.pallas_call(
        _project_body,
        grid=(NP // BLK,),
        in_specs=[
            pl.BlockSpec((BLK, D_FEAT), lambda i: (i, 0)),
            pl.BlockSpec((NUM_CLASSES, D_FEAT), lambda i: (0, 0)),
        ],
        out_specs=pl.BlockSpec((NUM_CLASSES, BLK), lambda i: (0, i)),
        out_shape=jax.ShapeDtypeStruct((NUM_CLASSES, NP), jnp.float32),
    )(x, W)


def _lsm_body(h_ref, b_ref, o_ref):
    p0 = h_ref[0:1, pl.ds(0, N_NODES)] + b_ref[0]
    p1 = h_ref[1:2, pl.ds(0, N_NODES)] + b_ref[1]
    m = jnp.maximum(p0, p1)
    lse = m + jnp.log(jnp.exp(p0 - m) + jnp.exp(p1 - m))
    o_ref[0:1, :] = p0 - lse
    o_ref[1:2, :] = p1 - lse


def _logsoftmax(h2, b):
    return pl.pallas_call(
        _lsm_body,
        in_specs=[
            pl.BlockSpec(memory_space=pltpu.VMEM),
            pl.BlockSpec(memory_space=pltpu.SMEM),
        ],
        out_shape=jax.ShapeDtypeStruct((NUM_CLASSES, N_NODES), jnp.float32),
    )(h2, b)


NSL = NP // NT
ECH = 20096


def _fill_f32(ref, val):
    v = jnp.full((16,), val, jnp.float32)

    @plsc.parallel_loop(0, NP // 16, unroll=8)
    def body(i):
        ref[pl.ds(i * 16, 16)] = v


def _sc_propagate(ei, y2):
    mesh = plsc.VectorSubcoreMesh(core_axis_name="c", subcore_axis_name="s")
    UN = 5

    @functools.partial(
        pl.kernel,
        mesh=mesh,
        compiler_params=pltpu.CompilerParams(needs_layout_passes=False),
        out_type=jax.ShapeDtypeStruct((NC, NP), jnp.float32),
        scratch_types=[
            pltpu.VMEM((2, ECH), jnp.int32),
            pltpu.VMEM((NP,), jnp.float32),
            pltpu.VMEM((NP,), jnp.float32),
            pltpu.VMEM((NP,), jnp.float32),
            pltpu.VMEM((NT, NSL), jnp.float32),
            pltpu.VMEM_SHARED((NT, NP), jnp.float32),
            pltpu.VMEM_SHARED((NP,), jnp.float32),
            pltpu.SemaphoreType.DMA,
            pltpu.SemaphoreType.DMA,
        ],
    )
    def sc_kernel(ei_hbm, y_hbm, out_hbm,
                  se_t, z, acc, dis, red_t, sh_all, sh_res, sem_e, sem_y):
        cid = lax.axis_index("c")
        sid = lax.axis_index("s")
        ebase = sid * EPT
        nbase = sid * NSL

        sh = (ebase % 128)
        abase = pl.multiple_of(ebase - sh, 128)
        cp_e = pltpu.async_copy(ei_hbm.at[:, pl.ds(abase, ECH)], se_t, sem_e)
        cp_y = pltpu.async_copy(y_hbm.at[cid], z, sem_y)

        def reduce_start(refill):
            pltpu.sync_copy(acc, sh_all.at[sid])
            plsc.subcore_barrier()
            cp_r = pltpu.async_copy(
                sh_all.at[:, pl.ds(nbase, NSL)], red_t, sem_e)
            if refill:
                _fill_f32(acc, 0.0)
            cp_r.wait()

        def sum_16(i):
            s = red_t[0, pl.ds(i * 16, 16)]
            for t in range(1, NT):
                s = s + red_t[t, pl.ds(i * 16, 16)]
            return s

        def publish_and_sync(dst_local):
            pltpu.sync_copy(red_t.at[0], sh_res.at[pl.ds(nbase, NSL)])
            plsc.subcore_barrier()
            pltpu.sync_copy(sh_res, dst_local)

        _fill_f32(acc, 0.0)
        ones16 = jnp.ones((16,), jnp.float32)
        cp_e.wait()

        @plsc.parallel_loop(0, EPT // 16, unroll=16)
        def deg_body(j):
            dv = se_t[1, pl.ds(sh + j * 16, 16)]
            plsc.addupdate_scatter(acc, [dv], ones16)

        reduce_start(refill=True)

        @plsc.parallel_loop(0, NSL // 16, unroll=2)
        def rs_body(i):
            d = sum_16(i) + 1.0
            bits = lax.bitcast_convert_type(d, jnp.int32)
            bits = 0x5F3759DF - (bits >> 1)
            x0 = lax.bitcast_convert_type(bits, jnp.float32)
            x0 = x0 * (1.5 - 0.5 * d * x0 * x0)
            x0 = x0 * (1.5 - 0.5 * d * x0 * x0)
            x0 = x0 * (1.5 - 0.5 * d * x0 * x0)
            red_t[0, pl.ds(i * 16, 16)] = x0

        publish_and_sync(dis)

        cp_y.wait()

        @plsc.parallel_loop(0, NP // 16, unroll=8)
        def z_body(i):
            o = i * 16
            z[pl.ds(o, 16)] = z[pl.ds(o, 16)] * dis[pl.ds(o, 16)]

        def hop(final):
            @plsc.parallel_loop(0, EPT // 16, unroll=16)
            def edge_body(j):
                o = sh + j * 16
                sv = se_t[0, pl.ds(o, 16)]
                dv = se_t[1, pl.ds(o, 16)]
                vals = plsc.load_gather(z, [sv])
                plsc.addupdate_scatter(acc, [dv], vals)

            reduce_start(refill=not final)

            @plsc.parallel_loop(0, NSL // 16, unroll=2)
            def sum_body(i):
                w = sum_16(i) + z[pl.ds(nbase + i * 16, 16)]
                d = dis[pl.ds(nbase + i * 16, 16)]
                scale = d if final else d * d
                red_t[0, pl.ds(i * 16, 16)] = w * scale

        hop(final=False)
        publish_and_sync(z)

        hop(final=True)
        pltpu.sync_copy(red_t.at[0], out_hbm.at[cid, pl.ds(nbase, NSL)])

    return sc_kernel(ei, y2)


def kernel(x, edge_index, W, b):
    ei = edge_index.astype(jnp.int32)
    yT = _project(x, W.T)
    h2 = _sc_propagate(ei, yT)
    out = _logsoftmax(h2, b)
    return out.T


# speedup vs baseline: 1.0219x; 1.0219x over previous
import functools

import jax
import jax.numpy as jnp
from jax import lax
from jax.experimental import pallas as pl
from jax.experimental.pallas import tpu as pltpu
from jax.experimental.pallas import tpu_sc as plsc

N_NODES = 10000
D_FEAT = 128
NUM_CLASSES = 2
N_EDGES = 320000

NP = 10240
ROWS = NP // 128
NT = 16
NC = 2
EPT = N_EDGES // NT
BLK = 2048


def _project_body(x_ref, w_ref, y_ref):
    y_ref[:, pl.ds(0, N_NODES)] = lax.dot_general(
        w_ref[...], x_ref[...],
        dimension_numbers=(((1,), (1,)), ((), ())),
        preferred_element_type=jnp.float32,
    )
    y_ref[:, pl.ds(N_NODES, NP - N_NODES)] = jnp.zeros(
        (NUM_CLASSES, NP - N_NODES), jnp.float32)


def _project(x, W):
    return pl.pallas_call(
        _project_body,
        out_shape=jax.ShapeDtypeStruct((NUM_CLASSES, NP), jnp.float32),
    )(x, W)


def _lsm_body(h_ref, b_ref, o_ref):
    p0 = h_ref[0:1, pl.ds(0, N_NODES)] + b_ref[0]
    p1 = h_ref[1:2, pl.ds(0, N_NODES)] + b_ref[1]
    m = jnp.maximum(p0, p1)
    lse = m + jnp.log(jnp.exp(p0 - m) + jnp.exp(p1 - m))
    o_ref[0:1, :] = p0 - lse
    o_ref[1:2, :] = p1 - lse


def _logsoftmax(h2, b):
    return pl.pallas_call(
        _lsm_body,
        in_specs=[
            pl.BlockSpec(memory_space=pltpu.VMEM),
            pl.BlockSpec(memory_space=pltpu.SMEM),
        ],
        out_shape=jax.ShapeDtypeStruct((NUM_CLASSES, N_NODES), jnp.float32),
    )(h2, b)


NSL = NP // NT
ECH = 20096


def _fill_f32(ref, val):
    v = jnp.full((16,), val, jnp.float32)

    @plsc.parallel_loop(0, NP // 16, unroll=8)
    def body(i):
        ref[pl.ds(i * 16, 16)] = v


def _sc_propagate(ei, y2):
    mesh = plsc.VectorSubcoreMesh(core_axis_name="c", subcore_axis_name="s")
    UN = 5

    @functools.partial(
        pl.kernel,
        mesh=mesh,
        compiler_params=pltpu.CompilerParams(needs_layout_passes=False),
        out_type=jax.ShapeDtypeStruct((NC, NP), jnp.float32),
        scratch_types=[
            pltpu.VMEM((2, ECH), jnp.int32),
            pltpu.VMEM((NP,), jnp.float32),
            pltpu.VMEM((NP,), jnp.float32),
            pltpu.VMEM((NP,), jnp.float32),
            pltpu.VMEM((NT, NSL), jnp.float32),
            pltpu.VMEM_SHARED((NT, NP), jnp.float32),
            pltpu.VMEM_SHARED((NP,), jnp.float32),
            pltpu.SemaphoreType.DMA,
            pltpu.SemaphoreType.DMA,
        ],
    )
    def sc_kernel(ei_hbm, y_hbm, out_hbm,
                  se_t, z, acc, dis, red_t, sh_all, sh_res, sem_e, sem_y):
        cid = lax.axis_index("c")
        sid = lax.axis_index("s")
        ebase = sid * EPT
        nbase = sid * NSL

        sh = (ebase % 128)
        abase = pl.multiple_of(ebase - sh, 128)
        cp_e = pltpu.async_copy(ei_hbm.at[:, pl.ds(abase, ECH)], se_t, sem_e)
        cp_y = pltpu.async_copy(y_hbm.at[cid], z, sem_y)

        def reduce_start(refill):
            pltpu.sync_copy(acc, sh_all.at[sid])
            plsc.subcore_barrier()
            cp_r = pltpu.async_copy(
                sh_all.at[:, pl.ds(nbase, NSL)], red_t, sem_e)
            if refill:
                _fill_f32(acc, 0.0)
            cp_r.wait()

        def sum_16(i):
            s = red_t[0, pl.ds(i * 16, 16)]
            for t in range(1, NT):
                s = s + red_t[t, pl.ds(i * 16, 16)]
            return s

        def publish_and_sync(dst_local):
            pltpu.sync_copy(red_t.at[0], sh_res.at[pl.ds(nbase, NSL)])
            plsc.subcore_barrier()
            pltpu.sync_copy(sh_res, dst_local)

        _fill_f32(acc, 0.0)
        ones16 = jnp.ones((16,), jnp.float32)
        cp_e.wait()

        @plsc.parallel_loop(0, EPT // 16, unroll=16)
        def deg_body(j):
            dv = se_t[1, pl.ds(sh + j * 16, 16)]
            plsc.addupdate_scatter(acc, [dv], ones16)

        reduce_start(refill=True)

        @plsc.parallel_loop(0, NSL // 16, unroll=2)
        def rs_body(i):
            d = sum_16(i) + 1.0
            bits = lax.bitcast_convert_type(d, jnp.int32)
            bits = 0x5F3759DF - (bits >> 1)
            x0 = lax.bitcast_convert_type(bits, jnp.float32)
            x0 = x0 * (1.5 - 0.5 * d * x0 * x0)
            x0 = x0 * (1.5 - 0.5 * d * x0 * x0)
            x0 = x0 * (1.5 - 0.5 * d * x0 * x0)
            red_t[0, pl.ds(i * 16, 16)] = x0

        publish_and_sync(dis)

        cp_y.wait()

        @plsc.parallel_loop(0, NP // 16, unroll=8)
        def z_body(i):
            o = i * 16
            z[pl.ds(o, 16)] = z[pl.ds(o, 16)] * dis[pl.ds(o, 16)]

        def hop(final):
            @plsc.parallel_loop(0, EPT // 16, unroll=16)
            def edge_body(j):
                o = sh + j * 16
                sv = se_t[0, pl.ds(o, 16)]
                dv = se_t[1, pl.ds(o, 16)]
                vals = plsc.load_gather(z, [sv])
                plsc.addupdate_scatter(acc, [dv], vals)

            reduce_start(refill=not final)

            @plsc.parallel_loop(0, NSL // 16, unroll=2)
            def sum_body(i):
                w = sum_16(i) + z[pl.ds(nbase + i * 16, 16)]
                d = dis[pl.ds(nbase + i * 16, 16)]
                scale = d if final else d * d
                red_t[0, pl.ds(i * 16, 16)] = w * scale

        hop(final=False)
        publish_and_sync(z)

        hop(final=True)
        pltpu.sync_copy(red_t.at[0], out_hbm.at[cid, pl.ds(nbase, NSL)])

    return sc_kernel(ei, y2)


def kernel(x, edge_index, W, b):
    ei = edge_index.astype(jnp.int32)
    yT = _project(x, W.T)
    h2 = _sc_propagate(ei, yT)
    out = _logsoftmax(h2, b)
    return out.T
